# baseline (speedup 1.0000x reference)
import functools

import jax
import jax.numpy as jnp
from jax import lax
from jax.experimental import pallas as pl
from jax.experimental.pallas import tpu as pltpu
from jax.experimental.pallas import tpu_sc as plsc


def _gather_kernel(B, T, D):
    info = plsc.get_sparse_core_info()
    NC, NS = info.num_cores, info.num_subcores
    NW = NC * NS
    assert B % (2 * NW) == 0
    rows_per_w = B // NW
    n2 = rows_per_w // 2

    mesh = plsc.VectorSubcoreMesh(core_axis_name="c", subcore_axis_name="s")

    @functools.partial(
        pl.kernel,
        mesh=mesh,
        out_type=jax.ShapeDtypeStruct((B, T, D), jnp.float32),
        scratch_types=[
            pltpu.VMEM((rows_per_w, T), jnp.int32),
            pltpu.VMEM((2, T, D), jnp.float32),
            pltpu.SemaphoreType.DMA,
            pltpu.SemaphoreType.DMA,
            pltpu.SemaphoreType.DMA,
            pltpu.SemaphoreType.DMA,
        ],
        compiler_params=pltpu.CompilerParams(use_tc_tiling_on_sc=False),
    )
    def k(idx_hbm, pe_hbm, out_hbm, idx_v, rows_v, sg0, sg1, so0, so1):
        wid = lax.axis_index("s") * NC + lax.axis_index("c")
        base = wid * rows_per_w
        pltpu.sync_copy(idx_hbm.at[pl.ds(base, rows_per_w)], idx_v)

        def gather_desc(r, buf, sem):
            return pltpu.make_async_copy(
                pe_hbm.at[idx_v.at[r]], rows_v.at[buf], sem
            )

        def store_desc(r, buf, sem):
            return pltpu.make_async_copy(
                rows_v.at[buf], out_hbm.at[base + r], sem
            )

        gather_desc(0, 0, sg0).start()
        gather_desc(1, 1, sg1).start()
        gather_desc(0, 0, sg0).wait()
        store_desc(0, 0, so0).start()
        gather_desc(1, 1, sg1).wait()
        store_desc(1, 1, so1).start()

        def body(p, carry):
            r0 = 2 * p
            store_desc(r0 - 2, 0, so0).wait()
            gather_desc(r0, 0, sg0).start()
            store_desc(r0 - 1, 1, so1).wait()
            gather_desc(r0 + 1, 1, sg1).start()
            gather_desc(r0, 0, sg0).wait()
            store_desc(r0, 0, so0).start()
            gather_desc(r0 + 1, 1, sg1).wait()
            store_desc(r0 + 1, 1, so1).start()
            return carry

        lax.fori_loop(1, n2, body, 0)
        store_desc(2 * n2 - 2, 0, so0).wait()
        store_desc(2 * n2 - 1, 1, so1).wait()

    return k


def kernel(idx, pe):
    B, T = idx.shape
    D = pe.shape[1]
    mid = _gather_kernel(B, T, D)(idx.astype(jnp.int32), pe)
    t1 = jnp.transpose(mid, (1, 2, 0))
    (t1,) = jax.lax.optimization_barrier((t1,))
    return jnp.transpose(t1, (2, 0, 1))


# speedup vs baseline: 1.1154x; 1.1154x over previous
import functools

import jax
import jax.numpy as jnp
from jax import lax
from jax.experimental import pallas as pl
from jax.experimental.pallas import tpu as pltpu
from jax.experimental.pallas import tpu_sc as plsc


def _gather_kernel(B, T, D):
    info = plsc.get_sparse_core_info()
    NC, NS = info.num_cores, info.num_subcores
    NW = NC * NS
    assert B % NW == 0
    rows_per_w = B // NW
    n_idx = rows_per_w * T

    mesh = plsc.VectorSubcoreMesh(core_axis_name="c", subcore_axis_name="s")

    @functools.partial(
        pl.kernel,
        mesh=mesh,
        out_type=jax.ShapeDtypeStruct((B, T, D), jnp.float32),
        scratch_types=[
            pltpu.VMEM((n_idx,), jnp.int32),
            pltpu.VMEM((2, T, 128), jnp.float32),
            pltpu.VMEM((2, T, D), jnp.float32),
            pltpu.SemaphoreType.DMA,
            pltpu.SemaphoreType.DMA,
            pltpu.SemaphoreType.DMA,
            pltpu.SemaphoreType.DMA,
        ],
    )
    def k(idx_hbm, pe_hbm, out_hbm, idx_v, wide_v, pack_v, sg0, sg1, so0, so1):
        wid = lax.axis_index("s") * NC + lax.axis_index("c")
        base = wid * rows_per_w
        pltpu.sync_copy(idx_hbm.at[pl.ds(base * T, n_idx)], idx_v)
        sg = (sg0, sg1)
        so = (so0, so1)

        def gather_desc(r, buf):
            return pltpu.make_async_copy(
                pe_hbm.at[idx_v.at[pl.ds(r * T, T)]],
                wide_v.at[buf],
                sg[buf],
            )

        def store_desc(r, buf):
            return pltpu.make_async_copy(
                pack_v.at[buf],
                out_hbm.at[base + r],
                so[buf],
            )

        def repack(buf):
            def rep_body(i4, carry):
                i0 = i4 * 4
                for rr in range(4):
                    for w in range(D // 16):
                        pack_v[buf, i0 + rr, pl.ds(w * 16, 16)] = wide_v[
                            buf, i0 + rr, pl.ds(w * 16, 16)
                        ]
                return carry

            lax.fori_loop(0, T // 4, rep_body, 0)

        gather_desc(0, 0).start()

        def body(r, carry):
            b = lax.rem(r, 2)

            def run(b):
                gather_desc(r, b).wait()

                @pl.when(r + 1 < rows_per_w)
                def _():
                    gather_desc(r + 1, 1 - b).start()

                @pl.when(r >= 2)
                def _():
                    store_desc(r - 2, b).wait()

                repack(b)
                store_desc(r, b).start()

            lax.cond(b == 0, lambda: run(0), lambda: run(1))
            return carry

        lax.fori_loop(0, rows_per_w, body, 0)
        store_desc(rows_per_w - 2, rows_per_w % 2).wait()
        store_desc(rows_per_w - 1, 1 - rows_per_w % 2).wait()

    return k


def kernel(idx, pe):
    B, T = idx.shape
    V, D = pe.shape
    flat_idx = idx.reshape(B * T).astype(jnp.int32)
    pe_pad = jnp.pad(pe, ((0, 0), (0, 128 - D)))
    return _gather_kernel(B, T, D)(flat_idx, pe_pad)


# speedup vs baseline: 1.1369x; 1.0193x over previous
import functools

import jax
import jax.numpy as jnp
from jax import lax
from jax.experimental import pallas as pl
from jax.experimental.pallas import tpu as pltpu
from jax.experimental.pallas import tpu_sc as plsc


def _gather_kernel(B, T, D):
    info = plsc.get_sparse_core_info()
    NC, NS = info.num_cores, info.num_subcores
    NW = NC * NS
    assert B % NW == 0
    rows_per_w = B // NW
    n_idx = rows_per_w * T

    mesh = plsc.VectorSubcoreMesh(core_axis_name="c", subcore_axis_name="s")

    @functools.partial(
        pl.kernel,
        mesh=mesh,
        out_type=jax.ShapeDtypeStruct((B, T, D), jnp.float32),
        scratch_types=[
            pltpu.VMEM((n_idx,), jnp.int32),
            pltpu.VMEM((2, T, 128), jnp.float32),
            pltpu.VMEM((2, T, D), jnp.float32),
            pltpu.SemaphoreType.DMA,
            pltpu.SemaphoreType.DMA,
            pltpu.SemaphoreType.DMA,
            pltpu.SemaphoreType.DMA,
        ],
    )
    def k(idx_hbm, pe_hbm, out_hbm, idx_v, wide_v, pack_v, sg0, sg1, so0, so1):
        wid = lax.axis_index("s") * NC + lax.axis_index("c")
        base = wid * rows_per_w
        pltpu.sync_copy(idx_hbm.at[pl.ds(base * T, n_idx)], idx_v)
        sg = (sg0, sg1)
        so = (so0, so1)

        def gather_desc(r, buf):
            return pltpu.make_async_copy(
                pe_hbm.at[idx_v.at[pl.ds(r * T, T)]],
                wide_v.at[buf],
                sg[buf],
            )

        def store_desc(r, buf):
            return pltpu.make_async_copy(
                pack_v.at[buf],
                out_hbm.at[base + r],
                so[buf],
            )

        def repack(buf):
            def rep_body(i4, carry):
                i0 = i4 * 4
                for rr in range(4):
                    for w in range(D // 16):
                        pack_v[buf, i0 + rr, pl.ds(w * 16, 16)] = wide_v[
                            buf, i0 + rr, pl.ds(w * 16, 16)
                        ]
                return carry

            lax.fori_loop(0, T // 4, rep_body, 0)

        gather_desc(0, 0).start()
        gather_desc(1, 1).start()

        def half(r, buf):
            gather_desc(r, buf).wait()

            @pl.when(r >= 2)
            def _():
                store_desc(r - 2, buf).wait()

            repack(buf)
            store_desc(r, buf).start()

            @pl.when(r + 2 < rows_per_w)
            def _():
                gather_desc(r + 2, buf).start()

        def body(p, carry):
            half(2 * p, 0)
            half(2 * p + 1, 1)
            return carry

        lax.fori_loop(0, rows_per_w // 2, body, 0)
        store_desc(rows_per_w - 2, 0).wait()
        store_desc(rows_per_w - 1, 1).wait()

    return k


def kernel(idx, pe):
    B, T = idx.shape
    V, D = pe.shape
    flat_idx = idx.reshape(B * T).astype(jnp.int32)
    pe_pad = jnp.pad(pe, ((0, 0), (0, 128 - D)))
    return _gather_kernel(B, T, D)(flat_idx, pe_pad)
